# baseline (speedup 1.0000x reference)
import jax
import jax.numpy as jnp
from jax.experimental import pallas as pl
from jax.experimental.pallas import tpu as pltpu

D_A = 768
KW, D_T = 8, 512
H, DH, FF = 12, 64, 3072
EPS = 1e-5
R = 16
MB = 4
KC = 128


def _ln(x, g, b):
    m = jnp.mean(x, axis=-1, keepdims=True)
    v = jnp.mean((x - m) ** 2, axis=-1, keepdims=True)
    return (x - m) * jax.lax.rsqrt(v + EPS) * g + b


def _bf(x):
    return x.astype(jnp.bfloat16)


def _encoder_kernel(a_ref, cls_ref, wq_ref, bq_ref, wkv_ref, bkv_ref,
                    wo_ref, bo_ref, ln1g_ref, ln1b_ref, w1_ref, b1_ref,
                    w2_ref, b2_ref, ln2g_ref, ln2b_ref, out_ref):
    t = a_ref.shape[1]
    cls16 = cls_ref[...]
    cls_bf = _bf(cls16)
    q = jnp.dot(cls_bf, wq_ref[...], preferred_element_type=jnp.float32) \
        + bq_ref[...]
    kv_c = _bf(jnp.dot(cls_bf, wkv_ref[...],
                       preferred_element_type=jnp.float32) + bkv_ref[...])
    a = a_ref[...].reshape(MB * t, D_A)
    kv_a = _bf(jnp.dot(a, wkv_ref[...],
                       preferred_element_type=jnp.float32) + bkv_ref[...])

    rows = H * R
    hm = (jax.lax.broadcasted_iota(jnp.int32, (rows, D_A), 0) // R
          == jax.lax.broadcasted_iota(jnp.int32, (rows, D_A), 1) // DH)
    q_bd = _bf(jnp.where(hm, jnp.concatenate([q] * H, axis=0), 0.0))
    zpad = jnp.zeros((KC - R, D_A), jnp.bfloat16)
    kc = jnp.concatenate([kv_c[:, :D_A], zpad], axis=0)
    vc = jnp.concatenate([kv_c[:, D_A:], zpad], axis=0)
    s_c = jax.lax.dot_general(q_bd, kc, (((1,), (1,)), ((), ())),
                              preferred_element_type=jnp.float32)
    cmask = jax.lax.broadcasted_iota(jnp.int32, (1, KC), 1) < (1 + KW)
    s_c = jnp.where(cmask, s_c, -1e30)
    sel = _bf(jax.lax.broadcasted_iota(jnp.int32, (R, rows), 0)
              == jax.lax.broadcasted_iota(jnp.int32, (R, rows), 1) % R)

    o16s = []
    for mb in range(MB):
        rsl = slice(mb * t, (mb + 1) * t)
        s_a = jax.lax.dot_general(q_bd, kv_a[rsl, :D_A],
                                  (((1,), (1,)), ((), ())),
                                  preferred_element_type=jnp.float32)
        s = jnp.concatenate([s_c, s_a], axis=1)
        m = jnp.max(s, axis=-1, keepdims=True)
        e = jnp.exp(s - m)
        den = jnp.sum(e, axis=-1, keepdims=True)
        p = _bf(e * (1.0 / den))
        o_full = (jnp.dot(p[:, :KC], vc, preferred_element_type=jnp.float32)
                  + jnp.dot(p[:, KC:], kv_a[rsl, D_A:],
                            preferred_element_type=jnp.float32))
        o_full = jnp.where(hm, o_full, 0.0)
        o16s.append(jnp.dot(sel, _bf(o_full),
                            preferred_element_type=jnp.float32))
    o = jnp.concatenate(o16s, axis=0)
    o = jnp.dot(_bf(o), wo_ref[...],
                preferred_element_type=jnp.float32) + bo_ref[...]
    xr = jnp.concatenate([cls16] * MB, axis=0)
    x1 = _ln(xr + o, ln1g_ref[...], ln1b_ref[...])
    hdn = jax.nn.gelu(jnp.dot(_bf(x1), w1_ref[...],
                              preferred_element_type=jnp.float32) + b1_ref[...])
    x2 = _ln(x1 + jnp.dot(_bf(hdn), w2_ref[...],
                          preferred_element_type=jnp.float32) + b2_ref[...],
             ln2g_ref[...], ln2b_ref[...])
    out_ref[...] = x2.reshape(MB, R, D_A)


def _vq_kernel(p_ref, kw_ref, pw_ref, pb_ref, cw_ref, cb_ref, bng_ref,
               bnb_ref, emb_ref, pout_ref, kwout_ref):
    bb = p_ref.shape[0]
    pout_ref[...] = (jnp.dot(_bf(p_ref[...]), pw_ref[...],
                             preferred_element_type=jnp.float32) + pb_ref[...])
    kw = (jnp.dot(_bf(kw_ref[...]), cw_ref[...],
                  preferred_element_type=jnp.float32) + cb_ref[...])
    kw3 = kw.reshape(bb, KW, D_T)
    mu = jnp.mean(kw3, axis=0, keepdims=True)
    var = jnp.mean((kw3 - mu) ** 2, axis=0, keepdims=True)
    kw3 = ((kw3 - mu) * jax.lax.rsqrt(var + EPS)
           * bng_ref[...][None] + bnb_ref[...][None])
    kw = kw3.reshape(bb * KW, D_T)
    kn = kw / (jnp.sqrt(jnp.sum(kw * kw, axis=-1, keepdims=True)) + 1e-8)
    emb = emb_ref[...]
    nsq = jnp.sum(emb * emb, axis=-1, keepdims=True)
    rn = 1.0 / (jnp.sqrt(nsq) + 1e-8)
    emb_bf = _bf(emb)
    cos = jax.lax.dot_general(_bf(kn), emb_bf, (((1,), (1,)), ((), ())),
                              preferred_element_type=jnp.float32)
    cos = cos * jnp.transpose(rn)
    e = jnp.exp(cos)
    den = jnp.sum(e, axis=-1, keepdims=True)
    kwout_ref[...] = jnp.dot(_bf(e), emb_bf,
                             preferred_element_type=jnp.float32) * (1.0 / den)


def kernel(audio_feat, params, token_emb):
    p = params
    bb, t, _ = audio_feat.shape
    cls16 = jnp.concatenate(
        [p['parallel_cls'][0], p['cascaded_cls'][0],
         jnp.zeros((R - 1 - KW, D_A), jnp.float32)], axis=0)
    scale = 1.0 / (DH ** 0.5)
    wkv = _bf(jnp.concatenate([p['Wk'], p['Wv']], axis=1))
    bkv = jnp.concatenate([p['bk'], p['bv']])[None]
    row = lambda a: a[None]

    full = lambda shp: pl.BlockSpec(shp, lambda i: (0,) * len(shp))
    x2 = pl.pallas_call(
        _encoder_kernel,
        grid=(bb // MB,),
        in_specs=[
            pl.BlockSpec((MB, t, D_A), lambda i: (i, 0, 0)),
            full((R, D_A)),
            full((D_A, D_A)), full((1, D_A)),
            full((D_A, 2 * D_A)), full((1, 2 * D_A)),
            full((D_A, D_A)), full((1, D_A)),
            full((1, D_A)), full((1, D_A)),
            full((D_A, FF)), full((1, FF)),
            full((FF, D_A)), full((1, D_A)),
            full((1, D_A)), full((1, D_A)),
        ],
        out_specs=pl.BlockSpec((MB, R, D_A), lambda i: (i, 0, 0)),
        out_shape=jax.ShapeDtypeStruct((bb, R, D_A), jnp.float32),
        compiler_params=pltpu.CompilerParams(
            dimension_semantics=("parallel",)),
    )(_bf(audio_feat), cls16, _bf(p['Wq'] * scale), row(p['bq'] * scale), wkv, bkv,
      _bf(p['Wo']), row(p['bo']), row(p['ln1_g']), row(p['ln1_b']),
      _bf(p['ffn_W1']), row(p['ffn_b1']), _bf(p['ffn_W2']), row(p['ffn_b2']),
      row(p['ln2_g']), row(p['ln2_b']))

    p_in = x2[:, 0, :]
    kw_in = x2[:, 1:1 + KW, :].reshape(bb * KW, D_A)
    vocab = token_emb.shape[0]

    pout, kwout = pl.pallas_call(
        _vq_kernel,
        in_specs=[
            pl.BlockSpec((bb, D_A), lambda: (0, 0)),
            pl.BlockSpec((bb * KW, D_A), lambda: (0, 0)),
            pl.BlockSpec((D_A, D_T), lambda: (0, 0)),
            pl.BlockSpec((1, D_T), lambda: (0, 0)),
            pl.BlockSpec((D_A, D_T), lambda: (0, 0)),
            pl.BlockSpec((1, D_T), lambda: (0, 0)),
            pl.BlockSpec((1, D_T), lambda: (0, 0)),
            pl.BlockSpec((1, D_T), lambda: (0, 0)),
            pl.BlockSpec((vocab, D_T), lambda: (0, 0)),
        ],
        out_specs=[
            pl.BlockSpec((bb, D_T), lambda: (0, 0)),
            pl.BlockSpec((bb * KW, D_T), lambda: (0, 0)),
        ],
        out_shape=[
            jax.ShapeDtypeStruct((bb, D_T), jnp.float32),
            jax.ShapeDtypeStruct((bb * KW, D_T), jnp.float32),
        ],
    )(p_in, kw_in, _bf(p['pproj_W']), row(p['pproj_b']), _bf(p['proj_W']),
      row(p['proj_b']), row(p['bn_g']), row(p['bn_b']), token_emb)

    return jnp.concatenate([pout[:, None, :], kwout.reshape(bb, KW, D_T)],
                           axis=1)


# speedup vs baseline: 1.2124x; 1.2124x over previous
import jax
import jax.numpy as jnp
from jax.experimental import pallas as pl
from jax.experimental.pallas import tpu as pltpu

D_A = 768
KW, D_T = 8, 512
H, DH, FF = 12, 64, 3072
EPS = 1e-5
R = 16
MB = 4
KC = 128


def _ln(x, g, b):
    m = jnp.mean(x, axis=-1, keepdims=True)
    v = jnp.mean((x - m) ** 2, axis=-1, keepdims=True)
    return (x - m) * jax.lax.rsqrt(v + EPS) * g + b


def _bf(x):
    return x.astype(jnp.bfloat16)


def _encoder_kernel(a_ref, cls_ref, wq_ref, bq_ref, wk_ref, bk_ref,
                    wv_ref, bv_ref, wo_ref, bo_ref, ln1g_ref, ln1b_ref,
                    w1_ref, b1_ref, w2_ref, b2_ref, ln2g_ref, ln2b_ref,
                    out_ref):
    t = a_ref.shape[1]
    scale = 1.0 / (DH ** 0.5)
    cls16 = cls_ref[...]
    q = (jnp.dot(cls16, wq_ref[...], preferred_element_type=jnp.float32)
         + bq_ref[...]) * scale
    k_c = _bf(jnp.dot(cls16, wk_ref[...],
                      preferred_element_type=jnp.float32) + bk_ref[...])
    v_c = _bf(jnp.dot(cls16, wv_ref[...],
                      preferred_element_type=jnp.float32) + bv_ref[...])
    a = a_ref[...].reshape(MB * t, D_A)
    k_a = _bf(jnp.dot(a, wk_ref[...],
                      preferred_element_type=jnp.float32) + bk_ref[...])
    v_a = _bf(jnp.dot(a, wv_ref[...],
                      preferred_element_type=jnp.float32) + bv_ref[...])

    rows = H * R
    hm = (jax.lax.broadcasted_iota(jnp.int32, (rows, D_A), 0) // R
          == jax.lax.broadcasted_iota(jnp.int32, (rows, D_A), 1) // DH)
    q_bd = _bf(jnp.where(hm, jnp.concatenate([q] * H, axis=0), 0.0))
    zpad = jnp.zeros((KC - R, D_A), jnp.bfloat16)
    kc = jnp.concatenate([k_c, zpad], axis=0)
    vc = jnp.concatenate([v_c, zpad], axis=0)
    s_c = jax.lax.dot_general(q_bd, kc, (((1,), (1,)), ((), ())),
                              preferred_element_type=jnp.float32)
    cmask = jax.lax.broadcasted_iota(jnp.int32, (1, KC), 1) < (1 + KW)
    s_c = jnp.where(cmask, s_c, -1e30)
    sel = _bf(jax.lax.broadcasted_iota(jnp.int32, (R, rows), 0)
              == jax.lax.broadcasted_iota(jnp.int32, (R, rows), 1) % R)

    o16s = []
    for mb in range(MB):
        rsl = slice(mb * t, (mb + 1) * t)
        s_a = jax.lax.dot_general(q_bd, k_a[rsl], (((1,), (1,)), ((), ())),
                                  preferred_element_type=jnp.float32)
        s = jnp.concatenate([s_c, s_a], axis=1)
        m = jnp.max(s, axis=-1, keepdims=True)
        e = jnp.exp(s - m)
        den = jnp.sum(e, axis=-1, keepdims=True)
        p = _bf(e * (1.0 / den))
        o_full = (jnp.dot(p[:, :KC], vc, preferred_element_type=jnp.float32)
                  + jnp.dot(p[:, KC:], v_a[rsl],
                            preferred_element_type=jnp.float32))
        o_full = jnp.where(hm, o_full, 0.0)
        o16s.append(jnp.dot(sel, _bf(o_full),
                            preferred_element_type=jnp.float32))
    o = jnp.concatenate(o16s, axis=0)
    o = jnp.dot(o, wo_ref[...],
                preferred_element_type=jnp.float32) + bo_ref[...]
    xr = jnp.concatenate([cls16] * MB, axis=0)
    x1 = _ln(xr + o, ln1g_ref[...], ln1b_ref[...])
    hdn = jax.nn.gelu(jnp.dot(x1, w1_ref[...],
                              preferred_element_type=jnp.float32) + b1_ref[...])
    x2 = _ln(x1 + jnp.dot(hdn, w2_ref[...],
                          preferred_element_type=jnp.float32) + b2_ref[...],
             ln2g_ref[...], ln2b_ref[...])
    out_ref[...] = x2.reshape(MB, R, D_A)


def _vq_kernel(p_ref, kw_ref, pw_ref, pb_ref, cw_ref, cb_ref, bng_ref,
               bnb_ref, emb_ref, pout_ref, kwout_ref):
    bb = p_ref.shape[0]
    pout_ref[...] = (jnp.dot(p_ref[...], pw_ref[...],
                             preferred_element_type=jnp.float32) + pb_ref[...])
    kw = (jnp.dot(kw_ref[...], cw_ref[...],
                  preferred_element_type=jnp.float32) + cb_ref[...])
    kw3 = kw.reshape(bb, KW, D_T)
    mu = jnp.mean(kw3, axis=0, keepdims=True)
    var = jnp.mean((kw3 - mu) ** 2, axis=0, keepdims=True)
    kw3 = ((kw3 - mu) * jax.lax.rsqrt(var + EPS)
           * bng_ref[...][None] + bnb_ref[...][None])
    kw = kw3.reshape(bb * KW, D_T)
    kn = kw / (jnp.sqrt(jnp.sum(kw * kw, axis=-1, keepdims=True)) + 1e-8)
    emb = emb_ref[...]
    nsq = jnp.sum(emb * emb, axis=-1, keepdims=True)
    rn = 1.0 / (jnp.sqrt(nsq) + 1e-8)
    cos = jax.lax.dot_general(kn, emb, (((1,), (1,)), ((), ())),
                              preferred_element_type=jnp.float32)
    cos = cos * jnp.transpose(rn)
    e = jnp.exp(cos)
    den = jnp.sum(e, axis=-1, keepdims=True)
    kwout_ref[...] = jnp.dot(e, emb,
                             preferred_element_type=jnp.float32) * (1.0 / den)


def kernel(audio_feat, params, token_emb):
    p = params
    bb, t, _ = audio_feat.shape
    cls16 = jnp.concatenate(
        [p['parallel_cls'][0], p['cascaded_cls'][0],
         jnp.zeros((R - 1 - KW, D_A), jnp.float32)], axis=0)
    row = lambda a: a[None]

    full = lambda shp: pl.BlockSpec(shp, lambda i: (0,) * len(shp))
    x2 = pl.pallas_call(
        _encoder_kernel,
        grid=(bb // MB,),
        in_specs=[
            pl.BlockSpec((MB, t, D_A), lambda i: (i, 0, 0)),
            full((R, D_A)),
            full((D_A, D_A)), full((1, D_A)),
            full((D_A, D_A)), full((1, D_A)),
            full((D_A, D_A)), full((1, D_A)),
            full((D_A, D_A)), full((1, D_A)),
            full((1, D_A)), full((1, D_A)),
            full((D_A, FF)), full((1, FF)),
            full((FF, D_A)), full((1, D_A)),
            full((1, D_A)), full((1, D_A)),
        ],
        out_specs=pl.BlockSpec((MB, R, D_A), lambda i: (i, 0, 0)),
        out_shape=jax.ShapeDtypeStruct((bb, R, D_A), jnp.float32),
        compiler_params=pltpu.CompilerParams(
            dimension_semantics=("parallel",)),
    )(audio_feat, cls16, p['Wq'], row(p['bq']), p['Wk'], row(p['bk']),
      p['Wv'], row(p['bv']), p['Wo'], row(p['bo']),
      row(p['ln1_g']), row(p['ln1_b']), p['ffn_W1'], row(p['ffn_b1']),
      p['ffn_W2'], row(p['ffn_b2']), row(p['ln2_g']), row(p['ln2_b']))

    p_in = x2[:, 0, :]
    kw_in = x2[:, 1:1 + KW, :].reshape(bb * KW, D_A)
    vocab = token_emb.shape[0]

    pout, kwout = pl.pallas_call(
        _vq_kernel,
        in_specs=[
            pl.BlockSpec((bb, D_A), lambda: (0, 0)),
            pl.BlockSpec((bb * KW, D_A), lambda: (0, 0)),
            pl.BlockSpec((D_A, D_T), lambda: (0, 0)),
            pl.BlockSpec((1, D_T), lambda: (0, 0)),
            pl.BlockSpec((D_A, D_T), lambda: (0, 0)),
            pl.BlockSpec((1, D_T), lambda: (0, 0)),
            pl.BlockSpec((1, D_T), lambda: (0, 0)),
            pl.BlockSpec((1, D_T), lambda: (0, 0)),
            pl.BlockSpec((vocab, D_T), lambda: (0, 0)),
        ],
        out_specs=[
            pl.BlockSpec((bb, D_T), lambda: (0, 0)),
            pl.BlockSpec((bb * KW, D_T), lambda: (0, 0)),
        ],
        out_shape=[
            jax.ShapeDtypeStruct((bb, D_T), jnp.float32),
            jax.ShapeDtypeStruct((bb * KW, D_T), jnp.float32),
        ],
    )(p_in, kw_in, p['pproj_W'], row(p['pproj_b']), p['proj_W'],
      row(p['proj_b']), row(p['bn_g']), row(p['bn_b']), token_emb)

    return jnp.concatenate([pout[:, None, :], kwout.reshape(bb, KW, D_T)],
                           axis=1)


# speedup vs baseline: 1.2141x; 1.0014x over previous
import jax
import jax.numpy as jnp
from jax.experimental import pallas as pl
from jax.experimental.pallas import tpu as pltpu

D_A = 768
KW, D_T = 8, 512
H, DH, FF = 12, 64, 3072
EPS = 1e-5
R = 16
MB = 4
KC = 128


def _ln(x, g, b):
    m = jnp.mean(x, axis=-1, keepdims=True)
    v = jnp.mean((x - m) ** 2, axis=-1, keepdims=True)
    return (x - m) * jax.lax.rsqrt(v + EPS) * g + b


def _bf(x):
    return x.astype(jnp.bfloat16)


def _encoder_kernel(a_ref, cls_ref, wq_ref, bq_ref, wk_ref, bk_ref,
                    wv_ref, bv_ref, wo_ref, bo_ref, ln1g_ref, ln1b_ref,
                    w1_ref, b1_ref, w2_ref, b2_ref, ln2g_ref, ln2b_ref,
                    out_ref):
    t = a_ref.shape[1]
    scale = 1.0 / (DH ** 0.5)
    cls16 = cls_ref[...]
    q = (jnp.dot(cls16, wq_ref[...], preferred_element_type=jnp.float32)
         + bq_ref[...]) * scale
    k_c = _bf(jnp.dot(cls16, wk_ref[...],
                      preferred_element_type=jnp.float32) + bk_ref[...])
    v_c = _bf(jnp.dot(cls16, wv_ref[...],
                      preferred_element_type=jnp.float32) + bv_ref[...])
    a = a_ref[...].reshape(MB * t, D_A)
    k_a = _bf(jnp.dot(a, wk_ref[...],
                      preferred_element_type=jnp.float32) + bk_ref[...])
    v_a = _bf(jnp.dot(a, wv_ref[...],
                      preferred_element_type=jnp.float32) + bv_ref[...])

    rows = H * R
    hm = (jax.lax.broadcasted_iota(jnp.int32, (rows, D_A), 0) // R
          == jax.lax.broadcasted_iota(jnp.int32, (rows, D_A), 1) // DH)
    q_bd = _bf(jnp.where(hm, jnp.concatenate([q] * H, axis=0), 0.0))
    zpad = jnp.zeros((KC - R, D_A), jnp.bfloat16)
    kc = jnp.concatenate([k_c, zpad], axis=0)
    vc = jnp.concatenate([v_c, zpad], axis=0)
    s_c = jax.lax.dot_general(q_bd, kc, (((1,), (1,)), ((), ())),
                              preferred_element_type=jnp.float32)
    cmask = jax.lax.broadcasted_iota(jnp.int32, (1, KC), 1) < (1 + KW)
    s_c = jnp.where(cmask, s_c, -1e30)
    sel = _bf(jax.lax.broadcasted_iota(jnp.int32, (R, rows), 0)
              == jax.lax.broadcasted_iota(jnp.int32, (R, rows), 1) % R)

    o16s = []
    for mb in range(MB):
        rsl = slice(mb * t, (mb + 1) * t)
        s_a = jax.lax.dot_general(q_bd, k_a[rsl], (((1,), (1,)), ((), ())),
                                  preferred_element_type=jnp.float32)
        s = jnp.concatenate([s_c, s_a], axis=1)
        m = jnp.max(s, axis=-1, keepdims=True)
        e = jnp.exp(s - m)
        den = jnp.sum(e, axis=-1, keepdims=True)
        p = _bf(e * (1.0 / den))
        o_full = (jnp.dot(p[:, :KC], vc, preferred_element_type=jnp.float32)
                  + jnp.dot(p[:, KC:], v_a[rsl],
                            preferred_element_type=jnp.float32))
        o_full = jnp.where(hm, o_full, 0.0)
        o16s.append(jnp.dot(sel, _bf(o_full),
                            preferred_element_type=jnp.float32))
    o = jnp.concatenate(o16s, axis=0)
    o = jnp.dot(o, wo_ref[...],
                preferred_element_type=jnp.float32) + bo_ref[...]
    xr = jnp.concatenate([cls16] * MB, axis=0)
    x1 = _ln(xr + o, ln1g_ref[...], ln1b_ref[...])
    hdn = jax.nn.gelu(jnp.dot(x1, w1_ref[...],
                              preferred_element_type=jnp.float32) + b1_ref[...])
    x2 = _ln(x1 + jnp.dot(hdn, w2_ref[...],
                          preferred_element_type=jnp.float32) + b2_ref[...],
             ln2g_ref[...], ln2b_ref[...])
    out_ref[...] = x2.reshape(MB, R, D_A)


def _vq_kernel(p_ref, kw_ref, pw_ref, pb_ref, cw_ref, cb_ref, bng_ref,
               bnb_ref, emb_ref, pout_ref, kwout_ref,
               kn_s, acc_s, den_s):
    bb = p_ref.shape[0]
    i = pl.program_id(0)
    nc = pl.num_programs(0)

    @pl.when(i == 0)
    def _head():
        pout_ref[...] = (jnp.dot(p_ref[...], pw_ref[...],
                                 preferred_element_type=jnp.float32)
                         + pb_ref[...])
        kw = (jnp.dot(kw_ref[...], cw_ref[...],
                      preferred_element_type=jnp.float32)
              + cb_ref[...])
        kw3 = kw.reshape(bb, KW, D_T)
        mu = jnp.mean(kw3, axis=0, keepdims=True)
        var = jnp.mean((kw3 - mu) ** 2, axis=0, keepdims=True)
        kw3 = ((kw3 - mu) * jax.lax.rsqrt(var + EPS)
               * bng_ref[...][None] + bnb_ref[...][None])
        kw = kw3.reshape(bb * KW, D_T)
        kn_s[...] = kw / (jnp.sqrt(jnp.sum(kw * kw, axis=-1, keepdims=True))
                          + 1e-8)
        acc_s[...] = jnp.zeros_like(acc_s)
        den_s[...] = jnp.zeros_like(den_s)

    emb = emb_ref[...]
    nsq = jnp.sum(emb * emb, axis=-1, keepdims=True)
    rn = 1.0 / (jnp.sqrt(nsq) + 1e-8)
    cos = jax.lax.dot_general(kn_s[...], emb, (((1,), (1,)), ((), ())),
                              preferred_element_type=jnp.float32)
    cos = cos * jnp.transpose(rn)
    e = jnp.exp(cos)
    den_s[...] += jnp.sum(e, axis=-1, keepdims=True)
    acc_s[...] += jnp.dot(e, emb, preferred_element_type=jnp.float32)

    @pl.when(i == nc - 1)
    def _tail():
        kwout_ref[...] = acc_s[...] * (1.0 / den_s[...])


def kernel(audio_feat, params, token_emb):
    p = params
    bb, t, _ = audio_feat.shape
    cls16 = jnp.concatenate(
        [p['parallel_cls'][0], p['cascaded_cls'][0],
         jnp.zeros((R - 1 - KW, D_A), jnp.float32)], axis=0)
    row = lambda a: a[None]

    full = lambda shp: pl.BlockSpec(shp, lambda i: (0,) * len(shp))
    x2 = pl.pallas_call(
        _encoder_kernel,
        grid=(bb // MB,),
        in_specs=[
            pl.BlockSpec((MB, t, D_A), lambda i: (i, 0, 0)),
            full((R, D_A)),
            full((D_A, D_A)), full((1, D_A)),
            full((D_A, D_A)), full((1, D_A)),
            full((D_A, D_A)), full((1, D_A)),
            full((D_A, D_A)), full((1, D_A)),
            full((1, D_A)), full((1, D_A)),
            full((D_A, FF)), full((1, FF)),
            full((FF, D_A)), full((1, D_A)),
            full((1, D_A)), full((1, D_A)),
        ],
        out_specs=pl.BlockSpec((MB, R, D_A), lambda i: (i, 0, 0)),
        out_shape=jax.ShapeDtypeStruct((bb, R, D_A), jnp.float32),
        compiler_params=pltpu.CompilerParams(
            dimension_semantics=("parallel",)),
    )(audio_feat, cls16, p['Wq'], row(p['bq']), p['Wk'], row(p['bk']),
      p['Wv'], row(p['bv']), p['Wo'], row(p['bo']),
      row(p['ln1_g']), row(p['ln1_b']), p['ffn_W1'], row(p['ffn_b1']),
      p['ffn_W2'], row(p['ffn_b2']), row(p['ln2_g']), row(p['ln2_b']))

    p_in = x2[:, 0, :]
    kw_in = x2[:, 1:1 + KW, :].reshape(bb * KW, D_A)
    vocab = token_emb.shape[0]

    vc = 1024
    pout, kwout = pl.pallas_call(
        _vq_kernel,
        grid=(vocab // vc,),
        in_specs=[
            pl.BlockSpec((bb, D_A), lambda i: (0, 0)),
            pl.BlockSpec((bb * KW, D_A), lambda i: (0, 0)),
            pl.BlockSpec((D_A, D_T), lambda i: (0, 0)),
            pl.BlockSpec((1, D_T), lambda i: (0, 0)),
            pl.BlockSpec((D_A, D_T), lambda i: (0, 0)),
            pl.BlockSpec((1, D_T), lambda i: (0, 0)),
            pl.BlockSpec((1, D_T), lambda i: (0, 0)),
            pl.BlockSpec((1, D_T), lambda i: (0, 0)),
            pl.BlockSpec((vc, D_T), lambda i: (i, 0)),
        ],
        out_specs=[
            pl.BlockSpec((bb, D_T), lambda i: (0, 0)),
            pl.BlockSpec((bb * KW, D_T), lambda i: (0, 0)),
        ],
        out_shape=[
            jax.ShapeDtypeStruct((bb, D_T), jnp.float32),
            jax.ShapeDtypeStruct((bb * KW, D_T), jnp.float32),
        ],
        scratch_shapes=[
            pltpu.VMEM((bb * KW, D_T), jnp.float32),
            pltpu.VMEM((bb * KW, D_T), jnp.float32),
            pltpu.VMEM((bb * KW, 1), jnp.float32),
        ],
    )(p_in, kw_in, p['pproj_W'], row(p['pproj_b']), p['proj_W'],
      row(p['proj_b']), row(p['bn_g']), row(p['bn_b']), token_emb)

    return jnp.concatenate([pout[:, None, :], kwout.reshape(bb, KW, D_T)],
                           axis=1)


# speedup vs baseline: 1.2650x; 1.0419x over previous
import jax
import jax.numpy as jnp
from jax.experimental import pallas as pl
from jax.experimental.pallas import tpu as pltpu

D_A = 768
KW, D_T = 8, 512
H, DH, FF = 12, 64, 3072
EPS = 1e-5
R = 16
MB = 4
KC = 128
NB = 4
VC = 1024


def _ln(x, g, b):
    m = jnp.mean(x, axis=-1, keepdims=True)
    v = jnp.mean((x - m) ** 2, axis=-1, keepdims=True)
    return (x - m) * jax.lax.rsqrt(v + EPS) * g + b


def _bf(x):
    return x.astype(jnp.bfloat16)


def _fused_kernel(a_ref, cls_ref, wq_ref, bq_ref, wk_ref, bk_ref,
                  wv_ref, bv_ref, wo_ref, bo_ref, ln1g_ref, ln1b_ref,
                  w1_ref, b1_ref, w2_ref, b2_ref, ln2g_ref, ln2b_ref,
                  pw_ref, pb_ref, cw_ref, cb_ref, bng_ref, bnb_ref,
                  emb_ref, pout_ref, kwout_ref, x2_s, kn_s, acc_s, den_s):
    t = a_ref.shape[1]
    i = pl.program_id(0)
    nsteps = pl.num_programs(0)

    @pl.when(i < NB)
    def _encoder():
        scale = 1.0 / (DH ** 0.5)
        cls16 = cls_ref[...]
        q = (jnp.dot(cls16, wq_ref[...], preferred_element_type=jnp.float32)
             + bq_ref[...]) * scale
        k_c = _bf(jnp.dot(cls16, wk_ref[...],
                          preferred_element_type=jnp.float32) + bk_ref[...])
        v_c = _bf(jnp.dot(cls16, wv_ref[...],
                          preferred_element_type=jnp.float32) + bv_ref[...])
        a = a_ref[...].reshape(MB * t, D_A)
        k_a = _bf(jnp.dot(a, wk_ref[...],
                          preferred_element_type=jnp.float32) + bk_ref[...])
        v_a = _bf(jnp.dot(a, wv_ref[...],
                          preferred_element_type=jnp.float32) + bv_ref[...])

        rows = H * R
        hm = (jax.lax.broadcasted_iota(jnp.int32, (rows, D_A), 0) // R
              == jax.lax.broadcasted_iota(jnp.int32, (rows, D_A), 1) // DH)
        q_bd = _bf(jnp.where(hm, jnp.concatenate([q] * H, axis=0), 0.0))
        zpad = jnp.zeros((KC - R, D_A), jnp.bfloat16)
        kc = jnp.concatenate([k_c, zpad], axis=0)
        vc = jnp.concatenate([v_c, zpad], axis=0)
        s_c = jax.lax.dot_general(q_bd, kc, (((1,), (1,)), ((), ())),
                                  preferred_element_type=jnp.float32)
        cmask = jax.lax.broadcasted_iota(jnp.int32, (1, KC), 1) < (1 + KW)
        s_c = jnp.where(cmask, s_c, -1e30)
        sel = _bf(jax.lax.broadcasted_iota(jnp.int32, (R, rows), 0)
                  == jax.lax.broadcasted_iota(jnp.int32, (R, rows), 1) % R)

        o16s = []
        for mb in range(MB):
            rsl = slice(mb * t, (mb + 1) * t)
            s_a = jax.lax.dot_general(q_bd, k_a[rsl],
                                      (((1,), (1,)), ((), ())),
                                      preferred_element_type=jnp.float32)
            s = jnp.concatenate([s_c, s_a], axis=1)
            m = jnp.max(s, axis=-1, keepdims=True)
            e = jnp.exp(s - m)
            den = jnp.sum(e, axis=-1, keepdims=True)
            p = _bf(e * (1.0 / den))
            o_full = (jnp.dot(p[:, :KC], vc,
                              preferred_element_type=jnp.float32)
                      + jnp.dot(p[:, KC:], v_a[rsl],
                                preferred_element_type=jnp.float32))
            o_full = jnp.where(hm, o_full, 0.0)
            o16s.append(jnp.dot(sel, _bf(o_full),
                                preferred_element_type=jnp.float32))
        o = jnp.concatenate(o16s, axis=0)
        o = jnp.dot(o, wo_ref[...],
                    preferred_element_type=jnp.float32) + bo_ref[...]
        xr = jnp.concatenate([cls16] * MB, axis=0)
        x1 = _ln(xr + o, ln1g_ref[...], ln1b_ref[...])
        hdn = jax.nn.gelu(jnp.dot(x1, w1_ref[...],
                                  preferred_element_type=jnp.float32)
                          + b1_ref[...])
        x2 = _ln(x1 + jnp.dot(hdn, w2_ref[...],
                              preferred_element_type=jnp.float32)
                 + b2_ref[...], ln2g_ref[...], ln2b_ref[...])
        x2_s[pl.ds(i * MB * R, MB * R), :] = x2

    @pl.when(i == NB)
    def _head():
        nrows = x2_s.shape[0]
        bb = nrows // R
        selp = (jax.lax.broadcasted_iota(jnp.int32, (bb, nrows), 1)
                == jax.lax.broadcasted_iota(jnp.int32, (bb, nrows), 0) * R
                ).astype(jnp.float32)
        rk = jax.lax.broadcasted_iota(jnp.int32, (bb * KW, nrows), 0)
        selk = (jax.lax.broadcasted_iota(jnp.int32, (bb * KW, nrows), 1)
                == (rk // KW) * R + rk % KW + 1).astype(jnp.float32)
        x2 = x2_s[...]
        p_in = jnp.dot(selp, x2, preferred_element_type=jnp.float32)
        kw_in = jnp.dot(selk, x2, preferred_element_type=jnp.float32)
        pout_ref[...] = (jnp.dot(p_in, pw_ref[...],
                                 preferred_element_type=jnp.float32)
                         + pb_ref[...])
        kw = (jnp.dot(kw_in, cw_ref[...],
                      preferred_element_type=jnp.float32)
              + cb_ref[...])
        kw3 = kw.reshape(bb, KW, D_T)
        mu = jnp.mean(kw3, axis=0, keepdims=True)
        var = jnp.mean((kw3 - mu) ** 2, axis=0, keepdims=True)
        kw3 = ((kw3 - mu) * jax.lax.rsqrt(var + EPS)
               * bng_ref[...][None] + bnb_ref[...][None])
        kw = kw3.reshape(bb * KW, D_T)
        kn_s[...] = kw / (jnp.sqrt(jnp.sum(kw * kw, axis=-1, keepdims=True))
                          + 1e-8)
        acc_s[...] = jnp.zeros_like(acc_s)
        den_s[...] = jnp.zeros_like(den_s)

    @pl.when(i >= NB)
    def _vq():
        emb = emb_ref[...]
        nsq = jnp.sum(emb * emb, axis=-1, keepdims=True)
        rn = 1.0 / (jnp.sqrt(nsq) + 1e-8)
        cos = jax.lax.dot_general(kn_s[...], emb, (((1,), (1,)), ((), ())),
                                  preferred_element_type=jnp.float32)
        cos = cos * jnp.transpose(rn)
        e = jnp.exp(cos)
        den_s[...] += jnp.sum(e, axis=-1, keepdims=True)
        acc_s[...] += jnp.dot(e, emb, preferred_element_type=jnp.float32)

    @pl.when(i == nsteps - 1)
    def _tail():
        kwout_ref[...] = acc_s[...] * (1.0 / den_s[...])


def kernel(audio_feat, params, token_emb):
    p = params
    bb, t, _ = audio_feat.shape
    vocab = token_emb.shape[0]
    nc = vocab // VC
    cls16 = jnp.concatenate(
        [p['parallel_cls'][0], p['cascaded_cls'][0],
         jnp.zeros((R - 1 - KW, D_A), jnp.float32)], axis=0)
    row = lambda a: a[None]

    full = lambda shp: pl.BlockSpec(shp, lambda i: (0,) * len(shp))
    pout, kwout = pl.pallas_call(
        _fused_kernel,
        grid=(NB + nc,),
        in_specs=[
            pl.BlockSpec((MB, t, D_A), lambda i: (jnp.minimum(i, NB - 1), 0, 0)),
            full((R, D_A)),
            full((D_A, D_A)), full((1, D_A)),
            full((D_A, D_A)), full((1, D_A)),
            full((D_A, D_A)), full((1, D_A)),
            full((D_A, D_A)), full((1, D_A)),
            full((1, D_A)), full((1, D_A)),
            full((D_A, FF)), full((1, FF)),
            full((FF, D_A)), full((1, D_A)),
            full((1, D_A)), full((1, D_A)),
            full((D_A, D_T)), full((1, D_T)),
            full((D_A, D_T)), full((1, D_T)),
            full((1, D_T)), full((1, D_T)),
            pl.BlockSpec((VC, D_T), lambda i: (jnp.maximum(i - NB, 0), 0)),
        ],
        out_specs=[
            pl.BlockSpec((bb, D_T), lambda i: (0, 0)),
            pl.BlockSpec((bb * KW, D_T), lambda i: (0, 0)),
        ],
        out_shape=[
            jax.ShapeDtypeStruct((bb, D_T), jnp.float32),
            jax.ShapeDtypeStruct((bb * KW, D_T), jnp.float32),
        ],
        scratch_shapes=[
            pltpu.VMEM((bb * R, D_A), jnp.float32),
            pltpu.VMEM((bb * KW, D_T), jnp.float32),
            pltpu.VMEM((bb * KW, D_T), jnp.float32),
            pltpu.VMEM((bb * KW, 1), jnp.float32),
        ],
        compiler_params=pltpu.CompilerParams(
            dimension_semantics=("arbitrary",),
            vmem_limit_bytes=100 * 1024 * 1024),
    )(audio_feat, cls16, p['Wq'], row(p['bq']), p['Wk'], row(p['bk']),
      p['Wv'], row(p['bv']), p['Wo'], row(p['bo']),
      row(p['ln1_g']), row(p['ln1_b']), p['ffn_W1'], row(p['ffn_b1']),
      p['ffn_W2'], row(p['ffn_b2']), row(p['ln2_g']), row(p['ln2_b']),
      p['pproj_W'], row(p['pproj_b']), p['proj_W'], row(p['proj_b']),
      row(p['bn_g']), row(p['bn_b']), token_emb)

    return jnp.concatenate([pout[:, None, :], kwout.reshape(bb, KW, D_T)],
                           axis=1)


# speedup vs baseline: 1.3055x; 1.0320x over previous
import jax
import jax.numpy as jnp
from jax.experimental import pallas as pl
from jax.experimental.pallas import tpu as pltpu

D_A = 768
KW, D_T = 8, 512
H, DH, FF = 12, 64, 3072
EPS = 1e-5
R = 16
MB = 4
KC = 128
NB = 4
NF = 8
FC = FF // NF
VC = 1024


def _ln(x, g, b):
    m = jnp.mean(x, axis=-1, keepdims=True)
    v = jnp.mean((x - m) ** 2, axis=-1, keepdims=True)
    return (x - m) * jax.lax.rsqrt(v + EPS) * g + b


def _bf(x):
    return x.astype(jnp.bfloat16)


def _fused_kernel(a_ref, cls_ref, wq_ref, bq_ref, wk_ref, bk_ref,
                  wv_ref, bv_ref, wo_ref, bo_ref, ln1g_ref, ln1b_ref,
                  w1_ref, b1_ref, w2_ref, b2_ref, ln2g_ref, ln2b_ref,
                  pw_ref, pb_ref, cw_ref, cb_ref, bng_ref, bnb_ref,
                  emb_ref, pout_ref, kwout_ref, x1_s, x2a_s, kn_s, acc_s,
                  den_s):
    t = a_ref.shape[1]
    i = pl.program_id(0)
    nsteps = pl.num_programs(0)

    @pl.when(i < NB)
    def _encoder():
        scale = 1.0 / (DH ** 0.5)
        cls16 = cls_ref[...]
        q = (jnp.dot(cls16, wq_ref[...], preferred_element_type=jnp.float32)
             + bq_ref[...]) * scale
        k_c = _bf(jnp.dot(cls16, wk_ref[...],
                          preferred_element_type=jnp.float32) + bk_ref[...])
        v_c = _bf(jnp.dot(cls16, wv_ref[...],
                          preferred_element_type=jnp.float32) + bv_ref[...])
        a = a_ref[...].reshape(MB * t, D_A)
        k_a = _bf(jnp.dot(a, wk_ref[...],
                          preferred_element_type=jnp.float32) + bk_ref[...])
        v_a = _bf(jnp.dot(a, wv_ref[...],
                          preferred_element_type=jnp.float32) + bv_ref[...])

        rows = H * R
        hm = (jax.lax.broadcasted_iota(jnp.int32, (rows, D_A), 0) // R
              == jax.lax.broadcasted_iota(jnp.int32, (rows, D_A), 1) // DH)
        q_bd = _bf(jnp.where(hm, jnp.concatenate([q] * H, axis=0), 0.0))
        zpad = jnp.zeros((KC - R, D_A), jnp.bfloat16)
        kc = jnp.concatenate([k_c, zpad], axis=0)
        vc = jnp.concatenate([v_c, zpad], axis=0)
        s_c = jax.lax.dot_general(q_bd, kc, (((1,), (1,)), ((), ())),
                                  preferred_element_type=jnp.float32)
        cmask = jax.lax.broadcasted_iota(jnp.int32, (1, KC), 1) < (1 + KW)
        s_c = jnp.where(cmask, s_c, -1e30)
        sel = _bf(jax.lax.broadcasted_iota(jnp.int32, (R, rows), 0)
                  == jax.lax.broadcasted_iota(jnp.int32, (R, rows), 1) % R)

        o16s = []
        for mb in range(MB):
            rsl = slice(mb * t, (mb + 1) * t)
            s_a = jax.lax.dot_general(q_bd, k_a[rsl],
                                      (((1,), (1,)), ((), ())),
                                      preferred_element_type=jnp.float32)
            s = jnp.concatenate([s_c, s_a], axis=1)
            m = jnp.max(s, axis=-1, keepdims=True)
            e = jnp.exp(s - m)
            den = jnp.sum(e, axis=-1, keepdims=True)
            p = _bf(e * (1.0 / den))
            o_full = (jnp.dot(p[:, :KC], vc,
                              preferred_element_type=jnp.float32)
                      + jnp.dot(p[:, KC:], v_a[rsl],
                                preferred_element_type=jnp.float32))
            o_full = jnp.where(hm, o_full, 0.0)
            o16s.append(jnp.dot(sel, _bf(o_full),
                                preferred_element_type=jnp.float32))
        o = jnp.concatenate(o16s, axis=0)
        o = jnp.dot(o, wo_ref[...],
                    preferred_element_type=jnp.float32) + bo_ref[...]
        xr = jnp.concatenate([cls16] * MB, axis=0)
        x1 = _ln(xr + o, ln1g_ref[...], ln1b_ref[...])
        x1_s[pl.ds(i * MB * R, MB * R), :] = x1

    @pl.when(jnp.logical_and(i >= NB, i < NB + NF))
    def _ffn():
        x1 = x1_s[...]
        hdn = jax.nn.gelu(jnp.dot(x1, w1_ref[...],
                                  preferred_element_type=jnp.float32)
                          + b1_ref[...])
        part = jnp.dot(hdn, w2_ref[...], preferred_element_type=jnp.float32)

        @pl.when(i == NB)
        def _init():
            x2a_s[...] = part

        @pl.when(i > NB)
        def _accum():
            x2a_s[...] += part

    @pl.when(i == NB + NF - 1)
    def _head():
        nrows = x1_s.shape[0]
        bb = nrows // R
        selp = (jax.lax.broadcasted_iota(jnp.int32, (bb, nrows), 1)
                == jax.lax.broadcasted_iota(jnp.int32, (bb, nrows), 0) * R
                ).astype(jnp.float32)
        rk = jax.lax.broadcasted_iota(jnp.int32, (bb * KW, nrows), 0)
        selk = (jax.lax.broadcasted_iota(jnp.int32, (bb * KW, nrows), 1)
                == (rk // KW) * R + rk % KW + 1).astype(jnp.float32)
        x2 = _ln(x1_s[...] + x2a_s[...] + b2_ref[...],
                 ln2g_ref[...], ln2b_ref[...])
        p_in = jnp.dot(selp, x2, preferred_element_type=jnp.float32)
        kw_in = jnp.dot(selk, x2, preferred_element_type=jnp.float32)
        pout_ref[...] = (jnp.dot(p_in, pw_ref[...],
                                 preferred_element_type=jnp.float32)
                         + pb_ref[...])
        kw = (jnp.dot(kw_in, cw_ref[...],
                      preferred_element_type=jnp.float32)
              + cb_ref[...])
        kw3 = kw.reshape(bb, KW, D_T)
        mu = jnp.mean(kw3, axis=0, keepdims=True)
        var = jnp.mean((kw3 - mu) ** 2, axis=0, keepdims=True)
        kw3 = ((kw3 - mu) * jax.lax.rsqrt(var + EPS)
               * bng_ref[...][None] + bnb_ref[...][None])
        kw = kw3.reshape(bb * KW, D_T)
        kn_s[...] = kw / (jnp.sqrt(jnp.sum(kw * kw, axis=-1, keepdims=True))
                          + 1e-8)
        acc_s[...] = jnp.zeros_like(acc_s)
        den_s[...] = jnp.zeros_like(den_s)

    @pl.when(i >= NB + NF)
    def _vq():
        emb = emb_ref[...]
        nsq = jnp.sum(emb * emb, axis=-1, keepdims=True)
        rn = 1.0 / (jnp.sqrt(nsq) + 1e-8)
        cos = jax.lax.dot_general(kn_s[...], emb, (((1,), (1,)), ((), ())),
                                  preferred_element_type=jnp.float32)
        cos = cos * jnp.transpose(rn)
        e = jnp.exp(cos)
        den_s[...] += jnp.sum(e, axis=-1, keepdims=True)
        acc_s[...] += jnp.dot(e, emb, preferred_element_type=jnp.float32)

    @pl.when(i == nsteps - 1)
    def _tail():
        kwout_ref[...] = acc_s[...] * (1.0 / den_s[...])


def kernel(audio_feat, params, token_emb):
    p = params
    bb, t, _ = audio_feat.shape
    vocab = token_emb.shape[0]
    nc = vocab // VC
    cls16 = jnp.concatenate(
        [p['parallel_cls'][0], p['cascaded_cls'][0],
         jnp.zeros((R - 1 - KW, D_A), jnp.float32)], axis=0)
    row = lambda a: a[None]

    full = lambda shp: pl.BlockSpec(shp, lambda i: (0,) * len(shp))
    pout, kwout = pl.pallas_call(
        _fused_kernel,
        grid=(NB + NF + nc,),
        in_specs=[
            pl.BlockSpec((MB, t, D_A), lambda i: (jnp.minimum(i, NB - 1), 0, 0)),
            full((R, D_A)),
            full((D_A, D_A)), full((1, D_A)),
            full((D_A, D_A)), full((1, D_A)),
            full((D_A, D_A)), full((1, D_A)),
            full((D_A, D_A)), full((1, D_A)),
            full((1, D_A)), full((1, D_A)),
            pl.BlockSpec((D_A, FC),
                         lambda i: (0, jnp.clip(i - NB, 0, NF - 1))),
            pl.BlockSpec((1, FC),
                         lambda i: (0, jnp.clip(i - NB, 0, NF - 1))),
            pl.BlockSpec((FC, D_A),
                         lambda i: (jnp.clip(i - NB, 0, NF - 1), 0)),
            full((1, D_A)),
            full((1, D_A)), full((1, D_A)),
            full((D_A, D_T)), full((1, D_T)),
            full((D_A, D_T)), full((1, D_T)),
            full((1, D_T)), full((1, D_T)),
            pl.BlockSpec((VC, D_T),
                         lambda i: (jnp.maximum(i - NB - NF, 0), 0)),
        ],
        out_specs=[
            pl.BlockSpec((bb, D_T), lambda i: (0, 0)),
            pl.BlockSpec((bb * KW, D_T), lambda i: (0, 0)),
        ],
        out_shape=[
            jax.ShapeDtypeStruct((bb, D_T), jnp.float32),
            jax.ShapeDtypeStruct((bb * KW, D_T), jnp.float32),
        ],
        scratch_shapes=[
            pltpu.VMEM((bb * R, D_A), jnp.float32),
            pltpu.VMEM((bb * R, D_A), jnp.float32),
            pltpu.VMEM((bb * KW, D_T), jnp.float32),
            pltpu.VMEM((bb * KW, D_T), jnp.float32),
            pltpu.VMEM((bb * KW, 1), jnp.float32),
        ],
        compiler_params=pltpu.CompilerParams(
            dimension_semantics=("arbitrary",),
            vmem_limit_bytes=100 * 1024 * 1024),
    )(audio_feat, cls16, p['Wq'], row(p['bq']), p['Wk'], row(p['bk']),
      p['Wv'], row(p['bv']), p['Wo'], row(p['bo']),
      row(p['ln1_g']), row(p['ln1_b']), p['ffn_W1'], row(p['ffn_b1']),
      p['ffn_W2'], row(p['ffn_b2']), row(p['ln2_g']), row(p['ln2_b']),
      p['pproj_W'], row(p['pproj_b']), p['proj_W'], row(p['proj_b']),
      row(p['bn_g']), row(p['bn_b']), token_emb)

    return jnp.concatenate([pout[:, None, :], kwout.reshape(bb, KW, D_T)],
                           axis=1)
